# baseline (speedup 1.0000x reference)
import jax
import jax.numpy as jnp
from jax import lax
from jax.experimental import pallas as pl
from jax.experimental.pallas import tpu as pltpu
from jax.experimental.pallas import tpu_sc as plsc

NC, NS = 2, 16
NW = NC * NS
D0, D1 = 32, 16
D = D0 + D1


def _sc_body(j_hbm, t0_hbm, t1_hbm, o_hbm, idx_all, rows0, rows1, tiles,
             gsem0, gsem1, gsem2, gsem3, wsem0, wsem1, wsem2, wsem3):
    wid = lax.axis_index("s") * NC + lax.axis_index("c")
    pairs_total = j_hbm.shape[0]
    ppw = pairs_total // NW
    units = ppw * 8
    gsems = (gsem0, gsem1, gsem2, gsem3)
    wsems = (wsem0, wsem1, wsem2, wsem3)

    pltpu.sync_copy(j_hbm.at[pl.ds(wid * ppw, ppw)], idx_all)

    iota = lax.iota(jnp.int32, 16)
    cvecs = []
    for h in range(3):
        c_vec = iota + (16 * h)
        cvecs.append((c_vec >> 3, c_vec & 7))

    def unit_coords(u):
        p = u >> 3
        ls = u & 7
        pg = wid * ppw + p
        lt = pg >> 5
        bb = pg & 31
        return p, ls, lt * 8 + ls, bb

    def fire_gathers(u, s):
        p, ls, _, _ = unit_coords(u)
        src = idx_all.at[p, ls]
        pltpu.async_copy(t0_hbm.at[src], rows0.at[s], gsems[s])
        pltpu.async_copy(t1_hbm.at[src], rows1.at[s], gsems[s])

    def wait_gathers(s):
        pltpu.make_async_copy(t0_hbm.at[idx_all.at[0, 0]], rows0.at[s],
                              gsems[s]).wait()
        pltpu.make_async_copy(t1_hbm.at[idx_all.at[0, 0]], rows1.at[s],
                              gsems[s]).wait()

    def transpose(s):
        t3 = tiles.at[s]

        @pl.loop(0, 128, unroll=16)
        def _row(bl):
            bls = jnp.full((16,), bl, jnp.int32)
            plsc.store_scatter(t3, [cvecs[0][0], cvecs[0][1], bls],
                               rows0[s, bl, pl.ds(0, 16)])
            plsc.store_scatter(t3, [cvecs[1][0], cvecs[1][1], bls],
                               rows0[s, bl, pl.ds(16, 16)])
            plsc.store_scatter(t3, [cvecs[2][0], cvecs[2][1], bls],
                               rows1[s, bl, pl.ds(0, 16)])

    def fire_write(u, s):
        _, _, l, bb = unit_coords(u)
        pltpu.async_copy(tiles.at[s, :, :, pl.ds(0, 128)],
                         o_hbm.at[l, :, bb], wsems[s])

    def drain_write(s):
        pltpu.make_async_copy(tiles.at[s, :, :, pl.ds(0, 128)],
                              o_hbm.at[0, :, 0], wsems[s]).wait()

    fire_gathers(0, 0)
    fire_gathers(1, 1)

    @pl.loop(0, units, step=4)
    def _quad(base):
        for k in range(4):
            u = base + k
            s = k
            s2 = (k + 2) % 4

            @pl.when(u + 2 < units)
            def _():
                fire_gathers(u + 2, s2)

            wait_gathers(s)

            @pl.when(u >= 4)
            def _():
                drain_write(s)

            transpose(s)
            fire_write(u, s)

    for s in range(4):
        drain_write(s)


def kernel(inputs, table0, table1):
    B, L = inputs.shape
    J = (inputs.T.reshape(L // 8, 8, B // 128, 128)
         .transpose(0, 2, 1, 3)
         .reshape((L // 8) * (B // 128), 8, 128))
    mesh = plsc.VectorSubcoreMesh(core_axis_name="c", subcore_axis_name="s")
    O = pl.kernel(
        _sc_body,
        out_type=jax.ShapeDtypeStruct((L, 6, B // 128, 8, 128), jnp.float32),
        mesh=mesh,
        compiler_params=pltpu.CompilerParams(use_tc_tiling_on_sc=False,
                                            needs_layout_passes=False),
        scratch_types=[
            pltpu.VMEM((25, 8, 128), jnp.int32),
            pltpu.VMEM((4, 128, D0), jnp.float32),
            pltpu.VMEM((4, 128, D1), jnp.float32),
            pltpu.VMEM((4, 6, 8, 133), jnp.float32),
        ] + [pltpu.SemaphoreType.DMA] * 8,
    )(J, table0, table1)
    return O.transpose(2, 4, 0, 1, 3).reshape(B, L, D)


# speedup vs baseline: 1.0279x; 1.0279x over previous
import jax
import jax.numpy as jnp
from jax import lax
from jax.experimental import pallas as pl
from jax.experimental.pallas import tpu as pltpu
from jax.experimental.pallas import tpu_sc as plsc

NC, NS = 2, 16
NW = NC * NS
D0, D1 = 32, 16
D = D0 + D1


def _sc_body(j_hbm, t0_hbm, t1_hbm, o_hbm, idx_all, rows0, rows1, tiles,
             gsem0, gsem1, gsem2, gsem3, wsem0, wsem1, wsem2, wsem3):
    wid = lax.axis_index("s") * NC + lax.axis_index("c")
    pairs_total = j_hbm.shape[0]
    ppw = pairs_total // NW
    units = ppw * 8
    gsems = (gsem0, gsem1, gsem2, gsem3)
    wsems = (wsem0, wsem1, wsem2, wsem3)

    pltpu.sync_copy(j_hbm.at[pl.ds(wid * ppw, ppw)], idx_all)

    iota = lax.iota(jnp.int32, 16)
    cvecs = []
    for h in range(3):
        c_vec = iota + (16 * h)
        cvecs.append((c_vec >> 3, c_vec & 7))

    def unit_coords(u):
        p = u >> 3
        ls = u & 7
        pg = wid * ppw + p
        lt = pg >> 5
        bb = pg & 31
        return p, ls, lt * 8 + ls, bb

    def fire_gathers(u, s):
        p, ls, _, _ = unit_coords(u)
        src = idx_all.at[p, ls]
        pltpu.async_copy(t0_hbm.at[src], rows0.at[s], gsems[s])
        pltpu.async_copy(t1_hbm.at[src], rows1.at[s], gsems[s])

    def wait_gathers(s):
        pltpu.make_async_copy(t0_hbm.at[idx_all.at[0, 0]], rows0.at[s],
                              gsems[s]).wait()
        pltpu.make_async_copy(t1_hbm.at[idx_all.at[0, 0]], rows1.at[s],
                              gsems[s]).wait()

    def transpose(s):
        t3 = tiles.at[s]

        @pl.loop(0, 128, unroll=8)
        def _row(bl):
            bls = jnp.full((16,), 0, jnp.int32) + bl
            plsc.store_scatter(t3, [cvecs[0][0], cvecs[0][1], bls],
                               rows0[s, bl, pl.ds(0, 16)])
            plsc.store_scatter(t3, [cvecs[1][0], cvecs[1][1], bls],
                               rows0[s, bl, pl.ds(16, 16)])
            plsc.store_scatter(t3, [cvecs[2][0], cvecs[2][1], bls],
                               rows1[s, bl, pl.ds(0, 16)])

    def fire_write(u, s):
        _, _, l, bb = unit_coords(u)
        pltpu.async_copy(tiles.at[s, :, :, pl.ds(0, 128)],
                         o_hbm.at[l, :, bb], wsems[s])

    def drain_write(s):
        pltpu.make_async_copy(tiles.at[s, :, :, pl.ds(0, 128)],
                              o_hbm.at[0, :, 0], wsems[s]).wait()

    fire_gathers(0, 0)
    fire_gathers(1, 1)

    @pl.loop(0, units, step=4)
    def _quad(base):
        for k in range(4):
            u = base + k
            s = k
            s2 = (k + 2) % 4

            @pl.when(u + 2 < units)
            def _():
                fire_gathers(u + 2, s2)

            wait_gathers(s)

            @pl.when(u >= 4)
            def _():
                drain_write(s)

            transpose(s)
            fire_write(u, s)

    for s in range(4):
        drain_write(s)


def kernel(inputs, table0, table1):
    B, L = inputs.shape
    J = (inputs.T.reshape(L // 8, 8, B // 128, 128)
         .transpose(0, 2, 1, 3)
         .reshape((L // 8) * (B // 128), 8, 128))
    mesh = plsc.VectorSubcoreMesh(core_axis_name="c", subcore_axis_name="s")
    O = pl.kernel(
        _sc_body,
        out_type=jax.ShapeDtypeStruct((L, 6, B // 128, 8, 128), jnp.float32),
        mesh=mesh,
        compiler_params=pltpu.CompilerParams(use_tc_tiling_on_sc=False,
                                            needs_layout_passes=False),
        scratch_types=[
            pltpu.VMEM((25, 8, 128), jnp.int32),
            pltpu.VMEM((4, 128, D0), jnp.float32),
            pltpu.VMEM((4, 128, D1), jnp.float32),
            pltpu.VMEM((4, 6, 8, 133), jnp.float32),
        ] + [pltpu.SemaphoreType.DMA] * 8,
    )(J, table0, table1)
    return O.transpose(2, 4, 0, 1, 3).reshape(B, L, D)
